# baseline (speedup 1.0000x reference)
import functools

import jax
import jax.numpy as jnp
from jax import lax
from jax.experimental import pallas as pl
from jax.experimental.pallas import tpu as pltpu
from jax.experimental.pallas import tpu_sc as plsc

F32 = jnp.float32
NC, NS, L = 2, 16, 16
NW = NC * NS
EB = 128


def _arc_pre_body(xa_ref, watt_ref, waa_ref, baa_ref, e_ref, em_ref, a_ref,
                  *, nreal):
    i = pl.program_id(0)
    x = xa_ref[...]
    br = x.shape[0]
    e = jnp.sum(x * watt_ref[...], axis=1, keepdims=True)
    rows = i * br + lax.broadcasted_iota(jnp.int32, (br, 1), 0)
    e = jnp.where(rows < nreal, e, -jnp.inf)
    e_ref[...] = e
    em_ref[...] = jnp.full((1, 8, 128), jnp.max(e), dtype=F32)
    a_ref[...] = (jnp.dot(x, waa_ref[...], preferred_element_type=F32)
                  + baa_ref[...])


def _arc_pre(xa_p, watt, waa_t, baa, nreal):
    ep, da = xa_p.shape
    oa = waa_t.shape[1]
    br = 2048
    g = ep // br
    return pl.pallas_call(
        functools.partial(_arc_pre_body, nreal=nreal),
        grid=(g,),
        in_specs=[
            pl.BlockSpec((br, da), lambda i: (i, 0)),
            pl.BlockSpec((1, da), lambda i: (0, 0)),
            pl.BlockSpec((da, oa), lambda i: (0, 0)),
            pl.BlockSpec((1, oa), lambda i: (0, 0)),
        ],
        out_specs=[
            pl.BlockSpec((br, 1), lambda i: (i, 0)),
            pl.BlockSpec((1, 8, 128), lambda i: (i, 0, 0)),
            pl.BlockSpec((br, oa), lambda i: (i, 0)),
        ],
        out_shape=[
            jax.ShapeDtypeStruct((ep, 1), F32),
            jax.ShapeDtypeStruct((g, 8, 128), F32),
            jax.ShapeDtypeStruct((ep, oa), F32),
        ],
    )(xa_p, watt, waa_t, baa)


def _node_pre_body(xv_ref, wvv_ref, bvv_ref, wav_ref, hs_ref, y_ref):
    x = xv_ref[...]
    hs_ref[...] = (jnp.dot(x, wvv_ref[...], preferred_element_type=F32)
                   + bvv_ref[...])
    y_ref[...] = jnp.dot(x, wav_ref[...], preferred_element_type=F32)


def _node_pre(x_v, wvv_t, bvv, wav_t):
    n, dv = x_v.shape
    ov = wvv_t.shape[1]
    oa = wav_t.shape[1]
    br = 2000
    return pl.pallas_call(
        _node_pre_body,
        grid=(n // br,),
        in_specs=[
            pl.BlockSpec((br, dv), lambda i: (i, 0)),
            pl.BlockSpec((dv, ov), lambda i: (0, 0)),
            pl.BlockSpec((1, ov), lambda i: (0, 0)),
            pl.BlockSpec((dv, oa), lambda i: (0, 0)),
        ],
        out_specs=[
            pl.BlockSpec((br, ov), lambda i: (i, 0)),
            pl.BlockSpec((br, oa), lambda i: (i, 0)),
        ],
        out_shape=[
            jax.ShapeDtypeStruct((n, ov), F32),
            jax.ShapeDtypeStruct((n, oa), F32),
        ],
    )(x_v, wvv_t, bvv, wav_t)


def _node_post_body(z_ref, hs_ref, wva_ref, hv_ref):
    h = (jnp.dot(z_ref[...], wva_ref[...], preferred_element_type=F32)
         + hs_ref[...])
    hv_ref[...] = jnp.maximum(h, 0.0)


def _node_post(z, h_self, wva_t):
    npad, da = z.shape
    n, ov = h_self.shape
    br = 2000
    return pl.pallas_call(
        _node_post_body,
        grid=(n // br,),
        in_specs=[
            pl.BlockSpec((br, da), lambda i: (i, 0)),
            pl.BlockSpec((br, ov), lambda i: (i, 0)),
            pl.BlockSpec((da, ov), lambda i: (0, 0)),
        ],
        out_specs=pl.BlockSpec((br, ov), lambda i: (i, 0)),
        out_shape=jax.ShapeDtypeStruct((n, ov), F32),
    )(z, h_self, wva_t)


def _mm_body(x_ref, w_ref, b_ref, o_ref):
    o_ref[...] = (jnp.dot(x_ref[...], w_ref[...], preferred_element_type=F32)
                  + b_ref[...])


def _mm_bias(x, w_t, b, br):
    r, k = x.shape
    o = w_t.shape[1]
    return pl.pallas_call(
        _mm_body,
        grid=(r // br,),
        in_specs=[
            pl.BlockSpec((br, k), lambda i: (i, 0)),
            pl.BlockSpec((k, o), lambda i: (0, 0)),
            pl.BlockSpec((1, o), lambda i: (0, 0)),
        ],
        out_specs=pl.BlockSpec((br, o), lambda i: (i, 0)),
        out_shape=jax.ShapeDtypeStruct((r, o), F32),
    )(x, w_t, b)


def _head_body(x_ref, w1_ref, b1_ref, w2_ref, b2_ref, wo_ref, bo_ref, o_ref):
    h = jnp.maximum(
        jnp.dot(x_ref[...], w1_ref[...], preferred_element_type=F32)
        + b1_ref[...], 0.0)
    h = jnp.maximum(
        jnp.dot(h, w2_ref[...], preferred_element_type=F32) + b2_ref[...],
        0.0)
    o_ref[...] = (jnp.dot(h, wo_ref[...], preferred_element_type=F32)
                  + bo_ref[...])


def _head(x, w1_t, b1, w2_t, b2, wo_t, bo):
    ep, k = x.shape
    h1 = w1_t.shape[1]
    h2 = w2_t.shape[1]
    o = wo_t.shape[1]
    br = 2048
    return pl.pallas_call(
        _head_body,
        grid=(ep // br,),
        in_specs=[
            pl.BlockSpec((br, k), lambda i: (i, 0)),
            pl.BlockSpec((k, h1), lambda i: (0, 0)),
            pl.BlockSpec((1, h1), lambda i: (0, 0)),
            pl.BlockSpec((h1, h2), lambda i: (0, 0)),
            pl.BlockSpec((1, h2), lambda i: (0, 0)),
            pl.BlockSpec((h2, o), lambda i: (0, 0)),
            pl.BlockSpec((1, o), lambda i: (0, 0)),
        ],
        out_specs=pl.BlockSpec((br, o), lambda i: (i, 0)),
        out_shape=jax.ShapeDtypeStruct((ep, o), F32),
    )(x, w1_t, b1, w2_t, b2, wo_t, bo)


def _sc_mesh():
    return plsc.VectorSubcoreMesh(core_axis_name="c", subcore_axis_name="s")


def _sc_denom(e_p, dst_p, gv, npad):
    ep = e_p.shape[0]
    ew = ep // NW
    ch = npad // NS

    @functools.partial(
        pl.kernel, mesh=_sc_mesh(),
        compiler_params=pltpu.CompilerParams(needs_layout_passes=False),
        out_type=jax.ShapeDtypeStruct((NC, npad), F32),
        scratch_types=[
            pltpu.VMEM((ew,), F32),
            pltpu.VMEM((ew,), jnp.int32),
            pltpu.VMEM((L,), F32),
            pltpu.VMEM((npad,), F32),
            pltpu.VMEM((ch,), F32),
            pltpu.VMEM((ch,), F32),
            pltpu.VMEM_SHARED((NS, npad), F32),
        ])
    def k(e_hbm, d_hbm, g_hbm, out_hbm, eb, db, gvb, acc, trow, tacc, shp):
        c = lax.axis_index("c")
        s = lax.axis_index("s")
        wid = s * NC + c
        base = wid * ew
        pltpu.sync_copy(e_hbm.at[pl.ds(base, ew)], eb)
        pltpu.sync_copy(d_hbm.at[pl.ds(base, ew)], db)
        pltpu.sync_copy(g_hbm, gvb)
        zero = jnp.zeros((L,), F32)

        @pl.loop(0, npad // L)
        def _(i):
            acc[pl.ds(i * L, L)] = zero

        g = gvb[...]

        @pl.loop(0, ew // L, unroll=4)
        def _(i):
            ev = eb[pl.ds(i * L, L)]
            iv = db[pl.ds(i * L, L)]
            vals = jnp.exp(ev - g)
            plsc.addupdate_scatter(acc, [iv], vals)

        pltpu.sync_copy(acc, shp.at[s])
        plsc.subcore_barrier()
        pltpu.sync_copy(shp.at[0, pl.ds(s * ch, ch)], tacc)
        for r in range(1, NS):
            pltpu.sync_copy(shp.at[r, pl.ds(s * ch, ch)], trow)

            @pl.loop(0, ch // L)
            def _(i):
                sl = pl.ds(i * L, L)
                tacc[sl] = tacc[sl] + trow[sl]

        pltpu.sync_copy(tacc, out_hbm.at[c, pl.ds(s * ch, ch)])

    return k(e_p, dst_p, gv)


def _sc_z(e_p, dst_p, gv, denom, xa_flat, da, npad):
    ep = e_p.shape[0]
    npass = 2
    dh = da // (NC * npass)
    ew = ep // NS
    zw = npad * dh
    ch = zw // NS

    @functools.partial(
        pl.kernel, mesh=_sc_mesh(),
        compiler_params=pltpu.CompilerParams(needs_layout_passes=False),
        out_type=jax.ShapeDtypeStruct((NC, npass, zw), F32),
        scratch_types=[
            pltpu.VMEM((ew,), F32),
            pltpu.VMEM((ew,), jnp.int32),
            pltpu.VMEM((L,), F32),
            pltpu.VMEM((npad,), F32),
            pltpu.VMEM((EB * da,), F32),
            pltpu.VMEM((zw,), F32),
            pltpu.VMEM((ch,), F32),
            pltpu.VMEM((ch,), F32),
            pltpu.VMEM_SHARED((NS, zw), F32),
        ])
    def k(e_hbm, d_hbm, g_hbm, dn_hbm, xa_hbm, out_hbm,
          eb, db, gvb, dloc, xab, accz, trow, tacc, shp):
        c = lax.axis_index("c")
        s = lax.axis_index("s")
        base = s * ew
        pltpu.sync_copy(e_hbm.at[pl.ds(base, ew)], eb)
        pltpu.sync_copy(d_hbm.at[pl.ds(base, ew)], db)
        pltpu.sync_copy(g_hbm, gvb)
        pltpu.sync_copy(dn_hbm, dloc)
        zero = jnp.zeros((L,), F32)
        g = gvb[...]
        iota = lax.iota(jnp.int32, L)

        for p in range(npass):
            @pl.loop(0, zw // L)
            def _(i):
                accz[pl.ds(i * L, L)] = zero

            c0 = c * (dh * npass) + p * dh

            @pl.loop(0, ew // EB)
            def _(b):
                pltpu.sync_copy(
                    xa_hbm.at[pl.ds((base + b * EB) * da, EB * da)], xab)

                @pl.loop(0, EB // L, unroll=2)
                def _(gi):
                    j0 = b * EB + gi * L
                    ev = eb[pl.ds(j0, L)]
                    dv = db[pl.ds(j0, L)]
                    vals = jnp.exp(ev - g)
                    den = plsc.load_gather(dloc, [dv])
                    att = jnp.where(den > 0.0, vals / den, 0.0)
                    xbase = (iota + gi * L) * da + c0
                    zbase = dv * dh
                    for cl in range(dh):
                        xv = plsc.load_gather(xab, [xbase + cl])
                        plsc.addupdate_scatter(accz, [zbase + cl],
                                               xv * att)

            for t in range(zw // ch):
                sl = pl.ds(t * ch, ch)
                pltpu.sync_copy(accz.at[sl], shp.at[s, sl])
            plsc.subcore_barrier()
            pltpu.sync_copy(shp.at[0, pl.ds(s * ch, ch)], tacc)
            for r in range(1, NS):
                pltpu.sync_copy(shp.at[r, pl.ds(s * ch, ch)], trow)

                @pl.loop(0, ch // L)
                def _(i):
                    sl = pl.ds(i * L, L)
                    tacc[sl] = tacc[sl] + trow[sl]

            pltpu.sync_copy(tacc, out_hbm.at[c, p, pl.ds(s * ch, ch)])
            plsc.subcore_barrier()

    return k(e_p, dst_p, gv, denom, xa_flat)


def _sc_ha(a_p, y, src_p, dst_p):
    ep, oa = a_p.shape
    ew = ep // NW
    nv = oa // L

    @functools.partial(
        pl.kernel, mesh=_sc_mesh(),
        compiler_params=pltpu.CompilerParams(needs_layout_passes=False),
        out_type=jax.ShapeDtypeStruct((ep, oa), F32),
        scratch_types=[
            pltpu.VMEM((EB,), jnp.int32),
            pltpu.VMEM((EB,), jnp.int32),
            pltpu.VMEM((EB, oa), F32),
            pltpu.VMEM((EB, oa), F32),
            pltpu.VMEM((EB, oa), F32),
            pltpu.SemaphoreType.DMA,
            pltpu.SemaphoreType.DMA,
        ])
    def k(a_hbm, y_hbm, s_hbm, d_hbm, out_hbm,
          isrc, idst, r1, r2, ab, sem1, sem2):
        c = lax.axis_index("c")
        s = lax.axis_index("s")
        wid = s * NC + c
        base = wid * ew

        @pl.loop(0, ew // EB)
        def _(b):
            off = base + b * EB
            pltpu.sync_copy(s_hbm.at[pl.ds(off, EB)], isrc)
            pltpu.sync_copy(d_hbm.at[pl.ds(off, EB)], idst)
            cp1 = pltpu.async_copy(y_hbm.at[isrc], r1, sem1)
            cp2 = pltpu.async_copy(y_hbm.at[idst], r2, sem2)
            pltpu.sync_copy(a_hbm.at[pl.ds(off, EB)], ab)
            cp1.wait()
            cp2.wait()

            @pl.loop(0, EB, unroll=4)
            def _(r):
                for v in range(nv):
                    sl = pl.ds(v * L, L)
                    ab[r, sl] = jnp.maximum(
                        ab[r, sl] + r1[r, sl] + r2[r, sl], 0.0)

            pltpu.sync_copy(ab, out_hbm.at[pl.ds(off, EB)])

    return k(a_p, y, src_p, dst_p)


def kernel(x_v, x_a, arc_index, W_vv1, b_vv1, W_va1, W_att1, b_att1, W_aa1,
           b_aa1, W_av1, W_vv2, b_vv2, W_va2, W_att2, b_att2, W_aa2, b_aa2,
           W_av2, W_d1, b_d1, W_d2, b_d2, W_o, b_o):
    n = x_v.shape[0]
    e, da = x_a.shape
    ew = -(-e // (NW * EB)) * EB
    ep = ew * NW
    npad = -(-n // (NS * L)) * (NS * L)

    arcp = jnp.pad(arc_index, ((0, 0), (0, ep - e)))
    srcp, dstp = arcp[0], arcp[1]
    xa1p = jnp.pad(x_a, ((0, ep - e), (0, 0)))

    e1, em1, a1 = _arc_pre(xa1p, W_att1, W_aa1.T, b_aa1.reshape(1, -1), e)
    hs1, y1 = _node_pre(x_v, W_vv1.T, b_vv1.reshape(1, -1), W_av1.T)
    g1 = jnp.full((L,), jnp.max(em1), F32)
    e1f = e1.reshape(ep)
    dnp = _sc_denom(e1f, dstp, g1, npad)
    dn = dnp[0] + dnp[1]
    zp = _sc_z(e1f, dstp, g1, dn, xa1p.reshape(-1), da, npad)
    zo = zp.reshape(NC, 2, npad, da // 4)
    z = jnp.concatenate([zo[0, 0], zo[0, 1], zo[1, 0], zo[1, 1]], axis=1)
    hv1 = _node_post(z, hs1, W_va1.T)
    ha1 = _sc_ha(a1, y1, srcp, dstp)

    a2 = _mm_bias(ha1, W_aa2.T, b_aa2.reshape(1, -1), 2048)
    y2 = _mm_bias(hv1, W_av2.T, jnp.zeros((1, W_av2.shape[0]), F32), 2000)
    ha2 = _sc_ha(a2, y2, srcp, dstp)

    out = _head(ha2, W_d1.T, b_d1.reshape(1, -1), W_d2.T,
                b_d2.reshape(1, -1), W_o.T, b_o.reshape(1, -1))
    return out[:e]


# speedup vs baseline: 1.1442x; 1.1442x over previous
import functools

import jax
import jax.numpy as jnp
from jax import lax
from jax.experimental import pallas as pl
from jax.experimental.pallas import tpu as pltpu
from jax.experimental.pallas import tpu_sc as plsc

F32 = jnp.float32
NC, NS, L = 2, 16, 16
NW = NC * NS
EB = 128


def _arc_pre_body(xa_ref, watt_ref, waa_ref, baa_ref, e_ref, em_ref, a_ref,
                  *, nreal):
    i = pl.program_id(0)
    x = xa_ref[...]
    br = x.shape[0]
    e = jnp.sum(x * watt_ref[...], axis=1, keepdims=True)
    rows = i * br + lax.broadcasted_iota(jnp.int32, (br, 1), 0)
    e = jnp.where(rows < nreal, e, -jnp.inf)
    e_ref[...] = e
    em_ref[...] = jnp.full((1, 8, 128), jnp.max(e), dtype=F32)
    a_ref[...] = (jnp.dot(x, waa_ref[...], preferred_element_type=F32)
                  + baa_ref[...])


def _arc_pre(xa_p, watt, waa_t, baa, nreal):
    ep, da = xa_p.shape
    oa = waa_t.shape[1]
    br = 2048
    g = ep // br
    return pl.pallas_call(
        functools.partial(_arc_pre_body, nreal=nreal),
        grid=(g,),
        in_specs=[
            pl.BlockSpec((br, da), lambda i: (i, 0)),
            pl.BlockSpec((1, da), lambda i: (0, 0)),
            pl.BlockSpec((da, oa), lambda i: (0, 0)),
            pl.BlockSpec((1, oa), lambda i: (0, 0)),
        ],
        out_specs=[
            pl.BlockSpec((br, 1), lambda i: (i, 0)),
            pl.BlockSpec((1, 8, 128), lambda i: (i, 0, 0)),
            pl.BlockSpec((br, oa), lambda i: (i, 0)),
        ],
        out_shape=[
            jax.ShapeDtypeStruct((ep, 1), F32),
            jax.ShapeDtypeStruct((g, 8, 128), F32),
            jax.ShapeDtypeStruct((ep, oa), F32),
        ],
    )(xa_p, watt, waa_t, baa)


def _node_pre_body(xv_ref, wvv_ref, bvv_ref, wav_ref, hs_ref, y_ref):
    x = xv_ref[...]
    hs_ref[...] = (jnp.dot(x, wvv_ref[...], preferred_element_type=F32)
                   + bvv_ref[...])
    y_ref[...] = jnp.dot(x, wav_ref[...], preferred_element_type=F32)


def _node_pre(x_v, wvv_t, bvv, wav_t):
    n, dv = x_v.shape
    ov = wvv_t.shape[1]
    oa = wav_t.shape[1]
    br = 2000
    return pl.pallas_call(
        _node_pre_body,
        grid=(n // br,),
        in_specs=[
            pl.BlockSpec((br, dv), lambda i: (i, 0)),
            pl.BlockSpec((dv, ov), lambda i: (0, 0)),
            pl.BlockSpec((1, ov), lambda i: (0, 0)),
            pl.BlockSpec((dv, oa), lambda i: (0, 0)),
        ],
        out_specs=[
            pl.BlockSpec((br, ov), lambda i: (i, 0)),
            pl.BlockSpec((br, oa), lambda i: (i, 0)),
        ],
        out_shape=[
            jax.ShapeDtypeStruct((n, ov), F32),
            jax.ShapeDtypeStruct((n, oa), F32),
        ],
    )(x_v, wvv_t, bvv, wav_t)


def _node_post_body(z_ref, hs_ref, wva_ref, hv_ref):
    h = (jnp.dot(z_ref[...], wva_ref[...], preferred_element_type=F32)
         + hs_ref[...])
    hv_ref[...] = jnp.maximum(h, 0.0)


def _node_post(z, h_self, wva_t):
    npad, da = z.shape
    n, ov = h_self.shape
    br = 2000
    return pl.pallas_call(
        _node_post_body,
        grid=(n // br,),
        in_specs=[
            pl.BlockSpec((br, da), lambda i: (i, 0)),
            pl.BlockSpec((br, ov), lambda i: (i, 0)),
            pl.BlockSpec((da, ov), lambda i: (0, 0)),
        ],
        out_specs=pl.BlockSpec((br, ov), lambda i: (i, 0)),
        out_shape=jax.ShapeDtypeStruct((n, ov), F32),
    )(z, h_self, wva_t)


def _mm_body(x_ref, w_ref, b_ref, o_ref):
    o_ref[...] = (jnp.dot(x_ref[...], w_ref[...], preferred_element_type=F32)
                  + b_ref[...])


def _mm_bias(x, w_t, b, br):
    r, k = x.shape
    o = w_t.shape[1]
    return pl.pallas_call(
        _mm_body,
        grid=(r // br,),
        in_specs=[
            pl.BlockSpec((br, k), lambda i: (i, 0)),
            pl.BlockSpec((k, o), lambda i: (0, 0)),
            pl.BlockSpec((1, o), lambda i: (0, 0)),
        ],
        out_specs=pl.BlockSpec((br, o), lambda i: (i, 0)),
        out_shape=jax.ShapeDtypeStruct((r, o), F32),
    )(x, w_t, b)


def _head_body(x_ref, w1_ref, b1_ref, w2_ref, b2_ref, wo_ref, bo_ref, o_ref):
    h = jnp.maximum(
        jnp.dot(x_ref[...], w1_ref[...], preferred_element_type=F32)
        + b1_ref[...], 0.0)
    h = jnp.maximum(
        jnp.dot(h, w2_ref[...], preferred_element_type=F32) + b2_ref[...],
        0.0)
    o_ref[...] = (jnp.dot(h, wo_ref[...], preferred_element_type=F32)
                  + bo_ref[...])


def _head(x, w1_t, b1, w2_t, b2, wo_t, bo):
    ep, k = x.shape
    h1 = w1_t.shape[1]
    h2 = w2_t.shape[1]
    o = wo_t.shape[1]
    br = 2048
    return pl.pallas_call(
        _head_body,
        grid=(ep // br,),
        in_specs=[
            pl.BlockSpec((br, k), lambda i: (i, 0)),
            pl.BlockSpec((k, h1), lambda i: (0, 0)),
            pl.BlockSpec((1, h1), lambda i: (0, 0)),
            pl.BlockSpec((h1, h2), lambda i: (0, 0)),
            pl.BlockSpec((1, h2), lambda i: (0, 0)),
            pl.BlockSpec((h2, o), lambda i: (0, 0)),
            pl.BlockSpec((1, o), lambda i: (0, 0)),
        ],
        out_specs=pl.BlockSpec((br, o), lambda i: (i, 0)),
        out_shape=jax.ShapeDtypeStruct((ep, o), F32),
    )(x, w1_t, b1, w2_t, b2, wo_t, bo)


def _sc_mesh():
    return plsc.VectorSubcoreMesh(core_axis_name="c", subcore_axis_name="s")


def _sc_denom(e_p, dst_p, gv, npad):
    ep = e_p.shape[0]
    ew = ep // NW
    ch = npad // NS

    @functools.partial(
        pl.kernel, mesh=_sc_mesh(),
        compiler_params=pltpu.CompilerParams(needs_layout_passes=False),
        out_type=jax.ShapeDtypeStruct((NC, npad), F32),
        scratch_types=[
            pltpu.VMEM((ew,), F32),
            pltpu.VMEM((ew,), jnp.int32),
            pltpu.VMEM((L,), F32),
            pltpu.VMEM((npad,), F32),
            pltpu.VMEM((ch,), F32),
            pltpu.VMEM((ch,), F32),
            pltpu.VMEM_SHARED((NS, npad), F32),
        ])
    def k(e_hbm, d_hbm, g_hbm, out_hbm, eb, db, gvb, acc, trow, tacc, shp):
        c = lax.axis_index("c")
        s = lax.axis_index("s")
        wid = s * NC + c
        base = wid * ew
        pltpu.sync_copy(e_hbm.at[pl.ds(base, ew)], eb)
        pltpu.sync_copy(d_hbm.at[pl.ds(base, ew)], db)
        pltpu.sync_copy(g_hbm, gvb)
        zero = jnp.zeros((L,), F32)

        @pl.loop(0, npad // L)
        def _(i):
            acc[pl.ds(i * L, L)] = zero

        g = gvb[...]

        @pl.loop(0, ew // L)
        def _(i):
            ev = eb[pl.ds(i * L, L)]
            iv = db[pl.ds(i * L, L)]
            vals = jnp.exp(ev - g)
            plsc.addupdate_scatter(acc, [iv], vals)

        pltpu.sync_copy(acc, shp.at[s])
        plsc.subcore_barrier()
        pltpu.sync_copy(shp.at[0, pl.ds(s * ch, ch)], tacc)
        for r in range(1, NS):
            pltpu.sync_copy(shp.at[r, pl.ds(s * ch, ch)], trow)

            @pl.loop(0, ch // L)
            def _(i):
                sl = pl.ds(i * L, L)
                tacc[sl] = tacc[sl] + trow[sl]

        pltpu.sync_copy(tacc, out_hbm.at[c, pl.ds(s * ch, ch)])

    return k(e_p, dst_p, gv)


def _sc_z(e_p, dst_p, gv, denom, xa_flat, da, npad):
    ep = e_p.shape[0]
    npass = 2
    dh = da // (NC * npass)
    ew = ep // NS
    zw = npad * dh
    ch = zw // NS

    @functools.partial(
        pl.kernel, mesh=_sc_mesh(),
        compiler_params=pltpu.CompilerParams(needs_layout_passes=False),
        out_type=jax.ShapeDtypeStruct((NC, npass, zw), F32),
        scratch_types=[
            pltpu.VMEM((ew,), F32),
            pltpu.VMEM((ew,), jnp.int32),
            pltpu.VMEM((L,), F32),
            pltpu.VMEM((npad,), F32),
            pltpu.VMEM((EB * da,), F32),
            pltpu.VMEM((zw,), F32),
            pltpu.VMEM((ch,), F32),
            pltpu.VMEM((ch,), F32),
            pltpu.VMEM_SHARED((NS, zw), F32),
        ])
    def k(e_hbm, d_hbm, g_hbm, dn_hbm, xa_hbm, out_hbm,
          eb, db, gvb, dloc, xab, accz, trow, tacc, shp):
        c = lax.axis_index("c")
        s = lax.axis_index("s")
        base = s * ew
        pltpu.sync_copy(e_hbm.at[pl.ds(base, ew)], eb)
        pltpu.sync_copy(d_hbm.at[pl.ds(base, ew)], db)
        pltpu.sync_copy(g_hbm, gvb)
        pltpu.sync_copy(dn_hbm, dloc)
        zero = jnp.zeros((L,), F32)
        g = gvb[...]
        iota = lax.iota(jnp.int32, L)

        for p in range(npass):
            @pl.loop(0, zw // L)
            def _(i):
                accz[pl.ds(i * L, L)] = zero

            c0 = c * (dh * npass) + p * dh

            @pl.loop(0, ew // EB)
            def _(b):
                pltpu.sync_copy(
                    xa_hbm.at[pl.ds((base + b * EB) * da, EB * da)], xab)

                @pl.loop(0, EB // L)
                def _(gi):
                    j0 = b * EB + gi * L
                    ev = eb[pl.ds(j0, L)]
                    dv = db[pl.ds(j0, L)]
                    vals = jnp.exp(ev - g)
                    den = plsc.load_gather(dloc, [dv])
                    att = jnp.where(den > 0.0, vals / den, 0.0)
                    xbase = (iota + gi * L) * da + c0
                    zbase = dv * dh
                    for cl in range(dh):
                        xv = plsc.load_gather(xab, [xbase + cl])
                        plsc.addupdate_scatter(accz, [zbase + cl],
                                               xv * att)

            for t in range(zw // ch):
                sl = pl.ds(t * ch, ch)
                pltpu.sync_copy(accz.at[sl], shp.at[s, sl])
            plsc.subcore_barrier()
            pltpu.sync_copy(shp.at[0, pl.ds(s * ch, ch)], tacc)
            for r in range(1, NS):
                pltpu.sync_copy(shp.at[r, pl.ds(s * ch, ch)], trow)

                @pl.loop(0, ch // L)
                def _(i):
                    sl = pl.ds(i * L, L)
                    tacc[sl] = tacc[sl] + trow[sl]

            pltpu.sync_copy(tacc, out_hbm.at[c, p, pl.ds(s * ch, ch)])
            plsc.subcore_barrier()

    return k(e_p, dst_p, gv, denom, xa_flat)


def _sc_ha(a_p, y, src_p, dst_p):
    ep, oa = a_p.shape
    ew = ep // NW
    nv = oa // L

    @functools.partial(
        pl.kernel, mesh=_sc_mesh(),
        compiler_params=pltpu.CompilerParams(needs_layout_passes=False),
        out_type=jax.ShapeDtypeStruct((ep, oa), F32),
        scratch_types=[
            pltpu.VMEM((EB,), jnp.int32),
            pltpu.VMEM((EB,), jnp.int32),
            pltpu.VMEM((EB, oa), F32),
            pltpu.VMEM((EB, oa), F32),
            pltpu.VMEM((EB, oa), F32),
            pltpu.SemaphoreType.DMA,
            pltpu.SemaphoreType.DMA,
        ])
    def k(a_hbm, y_hbm, s_hbm, d_hbm, out_hbm,
          isrc, idst, r1, r2, ab, sem1, sem2):
        c = lax.axis_index("c")
        s = lax.axis_index("s")
        wid = s * NC + c
        base = wid * ew

        @pl.loop(0, ew // EB)
        def _(b):
            off = base + b * EB
            pltpu.sync_copy(s_hbm.at[pl.ds(off, EB)], isrc)
            pltpu.sync_copy(d_hbm.at[pl.ds(off, EB)], idst)
            cp1 = pltpu.async_copy(y_hbm.at[isrc], r1, sem1)
            cp2 = pltpu.async_copy(y_hbm.at[idst], r2, sem2)
            pltpu.sync_copy(a_hbm.at[pl.ds(off, EB)], ab)
            cp1.wait()
            cp2.wait()

            @pl.loop(0, EB)
            def _(r):
                for v in range(nv):
                    sl = pl.ds(v * L, L)
                    ab[r, sl] = jnp.maximum(
                        ab[r, sl] + r1[r, sl] + r2[r, sl], 0.0)

            pltpu.sync_copy(ab, out_hbm.at[pl.ds(off, EB)])

    return k(a_p, y, src_p, dst_p)


def kernel(x_v, x_a, arc_index, W_vv1, b_vv1, W_va1, W_att1, b_att1, W_aa1,
           b_aa1, W_av1, W_vv2, b_vv2, W_va2, W_att2, b_att2, W_aa2, b_aa2,
           W_av2, W_d1, b_d1, W_d2, b_d2, W_o, b_o):
    n = x_v.shape[0]
    e, da = x_a.shape
    ew = -(-e // (NW * EB)) * EB
    ep = ew * NW
    npad = -(-n // (NS * L)) * (NS * L)

    arcp = jnp.pad(arc_index, ((0, 0), (0, ep - e)))
    srcp, dstp = arcp[0], arcp[1]
    xa1p = jnp.pad(x_a, ((0, ep - e), (0, 0)))

    e1, em1, a1 = _arc_pre(xa1p, W_att1, W_aa1.T, b_aa1.reshape(1, -1), e)
    hs1, y1 = _node_pre(x_v, W_vv1.T, b_vv1.reshape(1, -1), W_av1.T)
    g1 = jnp.full((L,), jnp.max(em1), F32)
    e1f = e1.reshape(ep)
    dnp = _sc_denom(e1f, dstp, g1, npad)
    dn = dnp[0] + dnp[1]
    zp = _sc_z(e1f, dstp, g1, dn, xa1p.reshape(-1), da, npad)
    zo = zp.reshape(NC, 2, npad, da // 4)
    z = jnp.concatenate([zo[0, 0], zo[0, 1], zo[1, 0], zo[1, 1]], axis=1)
    hv1 = _node_post(z, hs1, W_va1.T)
    ha1 = _sc_ha(a1, y1, srcp, dstp)

    a2 = _mm_bias(ha1, W_aa2.T, b_aa2.reshape(1, -1), 2048)
    y2 = _mm_bias(hv1, W_av2.T, jnp.zeros((1, W_av2.shape[0]), F32), 2000)
    ha2 = _sc_ha(a2, y2, srcp, dstp)

    out = _head(ha2, W_d1.T, b_d1.reshape(1, -1), W_d2.T,
                b_d2.reshape(1, -1), W_o.T, b_o.reshape(1, -1))
    return out[:e]
